# baseline (speedup 1.0000x reference)
import functools

import jax
import jax.numpy as jnp
import numpy as np
from jax import lax
from jax.experimental import pallas as pl

TEMPERATURE = 0.07
N_NEGATIVES = 32
_B = 64
_D = 64
_HW = 128 * 128
_N2 = 2 * _B


def _pool_body(f_ref, l_ref, t_ref, b_ref, c_ref):
    f = f_ref[0]
    l0 = l_ref[0, 0]
    l1 = l_ref[0, 1]
    t0 = jnp.sum(jnp.sum(f * l0[None, :, :], axis=2), axis=1)
    t1 = jnp.sum(jnp.sum(f * l1[None, :, :], axis=2), axis=1)
    s = jnp.sum(jnp.sum(f, axis=2), axis=1)
    t_ref[0, 0] = t0
    t_ref[0, 1] = t1
    b_ref[0, 0] = s - t0
    b_ref[0, 1] = s - t1
    c_ref[0, 0] = jnp.broadcast_to(jnp.sum(l0), (_D,))
    c_ref[0, 1] = jnp.broadcast_to(jnp.sum(l1), (_D,))


def _finale_body(t_ref, b_ref, cc_ref, cr_ref, tidc_ref, tidr_ref, p_ref,
                 out_ref):
    T = t_ref[...]
    Bg = b_ref[...]
    cntc = cc_ref[...]
    cntr = cr_ref[...]
    tidc = tidc_ref[...]
    tidr = tidr_ref[...]
    P = p_ref[...]

    rt = T / jnp.maximum(cntc, 1.0)
    rt = rt / jnp.maximum(
        jnp.sqrt(jnp.sum(rt * rt, axis=1, keepdims=True)), 1e-12)
    rb = Bg / jnp.maximum(float(_HW) - cntc, 1.0)
    rb = rb / jnp.maximum(
        jnp.sqrt(jnp.sum(rb * rb, axis=1, keepdims=True)), 1e-12)

    gt = lax.dot_general(rt, rt, (((1,), (1,)), ((), ())),
                         preferred_element_type=jnp.float32)
    gb = lax.dot_general(rt, rb, (((1,), (1,)), ((), ())),
                         preferred_element_type=jnp.float32)

    rowi = lax.broadcasted_iota(jnp.int32, (_N2, _N2), 0)
    colj = lax.broadcasted_iota(jnp.int32, (_N2, _N2), 1)
    tri = (rowi <= colj).astype(jnp.float32)

    cooc = (tidc != tidr) & (cntr != 0.0)
    cf = cooc.astype(jnp.float32)
    csum = lax.dot_general(cf, tri, (((1,), (0,)), ((), ())),
                           preferred_element_type=jnp.float32)
    ndiff = csum[:, _N2 - 1:_N2]
    jf = colj.astype(jnp.float32)
    key = jnp.where(cooc, csum - 1.0, ndiff + jf - csum)

    pcond = (tidc == tidr) & (colj != tidc)
    pf = pcond.astype(jnp.float32)
    psum = lax.dot_general(pf, tri, (((1,), (0,)), ((), ())),
                           preferred_element_type=jnp.float32)
    onehot = pf * (psum == 1.0).astype(jnp.float32)
    has_pos = psum[:, _N2 - 1:_N2] > 0.0
    fallback = (colj == 0).astype(jnp.float32)
    oh = jnp.where(has_pos, onehot, fallback)
    pos_logit = jnp.sum(oh * gt, axis=1, keepdims=True)

    cols = []
    for k in range(N_NEGATIVES):
        pk = P[:, k:k + 1].astype(jnp.float32)
        sel = pk < ndiff
        g = jnp.where(sel, gt, gb)
        match = (key == pk)
        cols.append(jnp.sum(jnp.where(match, g, 0.0), axis=1, keepdims=True))
    nl = jnp.concatenate(cols, axis=1) / TEMPERATURE
    pos = pos_logit / TEMPERATURE
    m = jnp.max(nl, axis=1, keepdims=True)
    row_loss = jnp.log(jnp.sum(jnp.exp(nl - m), axis=1, keepdims=True)) \
        - (pos - m)
    out_ref[...] = jnp.sum(row_loss, axis=0, keepdims=True) / float(_N2)


@functools.partial(jax.jit, static_argnames=())
def _run(features, labels, task_ids, perms):
    pool = pl.pallas_call(
        _pool_body,
        grid=(_B,),
        in_specs=[
            pl.BlockSpec((1, _D, 128, 128), lambda i: (i, 0, 0, 0)),
            pl.BlockSpec((1, 2, 128, 128), lambda i: (i, 0, 0, 0)),
        ],
        out_specs=[
            pl.BlockSpec((1, 2, _D), lambda i: (i, 0, 0)),
            pl.BlockSpec((1, 2, _D), lambda i: (i, 0, 0)),
            pl.BlockSpec((1, 2, _D), lambda i: (i, 0, 0)),
        ],
        out_shape=[
            jax.ShapeDtypeStruct((_B, 2, _D), jnp.float32),
            jax.ShapeDtypeStruct((_B, 2, _D), jnp.float32),
            jax.ShapeDtypeStruct((_B, 2, _D), jnp.float32),
        ],
    )
    t_sums, b_sums, cnts = pool(features, labels)

    t2 = t_sums.reshape(_N2, _D)
    b2 = b_sums.reshape(_N2, _D)
    cnt = cnts[:, :, 0].reshape(_N2)

    finale = pl.pallas_call(
        _finale_body,
        out_shape=jax.ShapeDtypeStruct((1, 1), jnp.float32),
    )
    loss = finale(
        t2, b2,
        cnt.reshape(_N2, 1), cnt.reshape(1, _N2),
        task_ids.reshape(_N2, 1), task_ids.reshape(1, _N2),
        perms,
    )
    return loss[0, 0]


_rng = np.random.default_rng(0)
_PERMS = jnp.asarray(
    np.stack([_rng.permutation(_D)[:N_NEGATIVES] for _ in range(_N2)]),
    dtype=jnp.int32)


def kernel(features, labels, tasks):
    task_ids = jnp.stack([2 * tasks, 2 * tasks + 1], axis=1).reshape(-1)
    return _run(features, labels, task_ids.astype(jnp.int32), _PERMS)


# speedup vs baseline: 1.6827x; 1.6827x over previous
import functools

import jax
import jax.numpy as jnp
import numpy as np
from jax import lax
from jax.experimental import pallas as pl

TEMPERATURE = 0.07
N_NEGATIVES = 32
_B = 64
_D = 64
_HW = 128 * 128
_N2 = 2 * _B


def _pool_body(f_ref, l_ref, tp_ref, sp_ref, c_ref):
    f = f_ref[0]
    l0 = l_ref[0, 0]
    l1 = l_ref[0, 1]
    tp_ref[0, 0] = jnp.sum(f * l0[None, :, :], axis=1)
    tp_ref[0, 1] = jnp.sum(f * l1[None, :, :], axis=1)
    s_part = jnp.sum(f, axis=1)
    sp_ref[0, 0] = s_part
    sp_ref[0, 1] = s_part
    c_ref[0, 0] = jnp.broadcast_to(jnp.sum(l0), (_D,))
    c_ref[0, 1] = jnp.broadcast_to(jnp.sum(l1), (_D,))


def _finale_body(tp_ref, sp_ref, cc_ref, cr_ref, tidc_ref, tidr_ref, p_ref,
                 out_ref):
    T = jnp.sum(tp_ref[...], axis=2)
    Bg = jnp.sum(sp_ref[...], axis=2) - T
    cntc = cc_ref[...]
    cntr = cr_ref[...]
    tidc = tidc_ref[...]
    tidr = tidr_ref[...]
    P = p_ref[...]

    rt = T / jnp.maximum(cntc, 1.0)
    rt = rt / jnp.maximum(
        jnp.sqrt(jnp.sum(rt * rt, axis=1, keepdims=True)), 1e-12)
    rb = Bg / jnp.maximum(float(_HW) - cntc, 1.0)
    rb = rb / jnp.maximum(
        jnp.sqrt(jnp.sum(rb * rb, axis=1, keepdims=True)), 1e-12)

    gt = lax.dot_general(rt, rt, (((1,), (1,)), ((), ())),
                         preferred_element_type=jnp.float32)
    gb = lax.dot_general(rt, rb, (((1,), (1,)), ((), ())),
                         preferred_element_type=jnp.float32)

    rowi = lax.broadcasted_iota(jnp.int32, (_N2, _N2), 0)
    colj = lax.broadcasted_iota(jnp.int32, (_N2, _N2), 1)
    tri = (rowi <= colj).astype(jnp.float32)

    cooc = (tidc != tidr) & (cntr != 0.0)
    cf = cooc.astype(jnp.float32)
    csum = lax.dot_general(cf, tri, (((1,), (0,)), ((), ())),
                           preferred_element_type=jnp.float32)
    ndiff = csum[:, _N2 - 1:_N2]
    jf = colj.astype(jnp.float32)
    key = jnp.where(cooc, csum - 1.0, ndiff + jf - csum)

    pcond = (tidc == tidr) & (colj != tidc)
    pf = pcond.astype(jnp.float32)
    psum = lax.dot_general(pf, tri, (((1,), (0,)), ((), ())),
                           preferred_element_type=jnp.float32)
    onehot = pf * (psum == 1.0).astype(jnp.float32)
    has_pos = psum[:, _N2 - 1:_N2] > 0.0
    fallback = (colj == 0).astype(jnp.float32)
    oh = jnp.where(has_pos, onehot, fallback)
    pos_logit = jnp.sum(oh * gt, axis=1, keepdims=True)

    cols = []
    for k in range(N_NEGATIVES):
        pk = P[:, k:k + 1].astype(jnp.float32)
        sel = pk < ndiff
        g = jnp.where(sel, gt, gb)
        match = (key == pk)
        cols.append(jnp.sum(jnp.where(match, g, 0.0), axis=1, keepdims=True))
    nl = jnp.concatenate(cols, axis=1) / TEMPERATURE
    pos = pos_logit / TEMPERATURE
    m = jnp.max(nl, axis=1, keepdims=True)
    row_loss = jnp.log(jnp.sum(jnp.exp(nl - m), axis=1, keepdims=True)) \
        - (pos - m)
    out_ref[...] = jnp.sum(row_loss, axis=0, keepdims=True) / float(_N2)


@functools.partial(jax.jit, static_argnames=())
def _run(features, labels, task_ids, perms):
    pool = pl.pallas_call(
        _pool_body,
        grid=(_B,),
        in_specs=[
            pl.BlockSpec((1, _D, 128, 128), lambda i: (i, 0, 0, 0)),
            pl.BlockSpec((1, 2, 128, 128), lambda i: (i, 0, 0, 0)),
        ],
        out_specs=[
            pl.BlockSpec((1, 2, _D, 128), lambda i: (i, 0, 0, 0)),
            pl.BlockSpec((1, 2, _D, 128), lambda i: (i, 0, 0, 0)),
            pl.BlockSpec((1, 2, _D), lambda i: (i, 0, 0)),
        ],
        out_shape=[
            jax.ShapeDtypeStruct((_B, 2, _D, 128), jnp.float32),
            jax.ShapeDtypeStruct((_B, 2, _D, 128), jnp.float32),
            jax.ShapeDtypeStruct((_B, 2, _D), jnp.float32),
        ],
    )
    t_parts, s_parts, cnts = pool(features, labels)

    t2 = t_parts.reshape(_N2, _D, 128)
    b2 = s_parts.reshape(_N2, _D, 128)
    cnt = cnts[:, :, 0].reshape(_N2)

    finale = pl.pallas_call(
        _finale_body,
        out_shape=jax.ShapeDtypeStruct((1, 1), jnp.float32),
    )
    loss = finale(
        t2, b2,
        cnt.reshape(_N2, 1), cnt.reshape(1, _N2),
        task_ids.reshape(_N2, 1), task_ids.reshape(1, _N2),
        perms,
    )
    return loss[0, 0]


_rng = np.random.default_rng(0)
_PERMS = np.stack(
    [_rng.permutation(_D)[:N_NEGATIVES] for _ in range(_N2)]).astype(np.int32)


def kernel(features, labels, tasks):
    task_ids = jnp.stack([2 * tasks, 2 * tasks + 1], axis=1).reshape(-1)
    return _run(features, labels, task_ids.astype(jnp.int32), _PERMS)


# speedup vs baseline: 1.9365x; 1.1508x over previous
import functools

import jax
import jax.numpy as jnp
import numpy as np
from jax import lax
from jax.experimental import pallas as pl

TEMPERATURE = 0.07
N_NEGATIVES = 32
_B = 64
_D = 64
_HW = 128 * 128
_N2 = 2 * _B


def _pool_body(f_ref, l_ref, t_ref, b_ref, c_ref):
    f = f_ref[0]
    l0 = l_ref[0, 0]
    l1 = l_ref[0, 1]
    t0p = jnp.sum(f * l0[None, :, :], axis=1)
    t1p = jnp.sum(f * l1[None, :, :], axis=1)
    sp = jnp.sum(f, axis=1)
    t0 = jnp.sum(t0p, axis=1)
    t1 = jnp.sum(t1p, axis=1)
    s = jnp.sum(sp, axis=1)
    t_ref[0, 0] = t0
    t_ref[0, 1] = t1
    b_ref[0, 0] = s - t0
    b_ref[0, 1] = s - t1
    c_ref[0, 0] = jnp.broadcast_to(jnp.sum(l0), (_D,))
    c_ref[0, 1] = jnp.broadcast_to(jnp.sum(l1), (_D,))


def _finale_body(t_ref, b_ref, cc_ref, cr_ref, tidc_ref, tidr_ref, p_ref,
                 out_ref):
    T = t_ref[...]
    Bg = b_ref[...]
    cntc = cc_ref[...]
    cntr = cr_ref[...]
    tidc = tidc_ref[...]
    tidr = tidr_ref[...]
    P = p_ref[...]

    rt = T / jnp.maximum(cntc, 1.0)
    rt = rt / jnp.maximum(
        jnp.sqrt(jnp.sum(rt * rt, axis=1, keepdims=True)), 1e-12)
    rb = Bg / jnp.maximum(float(_HW) - cntc, 1.0)
    rb = rb / jnp.maximum(
        jnp.sqrt(jnp.sum(rb * rb, axis=1, keepdims=True)), 1e-12)

    gt = lax.dot_general(rt, rt, (((1,), (1,)), ((), ())),
                         preferred_element_type=jnp.float32)
    gb = lax.dot_general(rt, rb, (((1,), (1,)), ((), ())),
                         preferred_element_type=jnp.float32)

    rowi = lax.broadcasted_iota(jnp.int32, (_N2, _N2), 0)
    colj = lax.broadcasted_iota(jnp.int32, (_N2, _N2), 1)
    tri = (rowi <= colj).astype(jnp.float32)

    cooc = (tidc != tidr) & (cntr != 0.0)
    cf = cooc.astype(jnp.float32)
    csum = lax.dot_general(cf, tri, (((1,), (0,)), ((), ())),
                           preferred_element_type=jnp.float32)
    ndiff = csum[:, _N2 - 1:_N2]
    jf = colj.astype(jnp.float32)
    key = jnp.where(cooc, csum - 1.0, ndiff + jf - csum)

    pcond = (tidc == tidr) & (colj != tidc)
    pf = pcond.astype(jnp.float32)
    psum = lax.dot_general(pf, tri, (((1,), (0,)), ((), ())),
                           preferred_element_type=jnp.float32)
    onehot = pf * (psum == 1.0).astype(jnp.float32)
    has_pos = psum[:, _N2 - 1:_N2] > 0.0
    fallback = (colj == 0).astype(jnp.float32)
    oh = jnp.where(has_pos, onehot, fallback)
    pos_logit = jnp.sum(oh * gt, axis=1, keepdims=True)

    cols = []
    for k in range(N_NEGATIVES):
        pk = P[:, k:k + 1].astype(jnp.float32)
        sel = pk < ndiff
        g = jnp.where(sel, gt, gb)
        match = (key == pk)
        cols.append(jnp.sum(jnp.where(match, g, 0.0), axis=1, keepdims=True))
    nl = jnp.concatenate(cols, axis=1) / TEMPERATURE
    pos = pos_logit / TEMPERATURE
    m = jnp.max(nl, axis=1, keepdims=True)
    row_loss = jnp.log(jnp.sum(jnp.exp(nl - m), axis=1, keepdims=True)) \
        - (pos - m)
    out_ref[...] = jnp.sum(row_loss, axis=0, keepdims=True) / float(_N2)


@functools.partial(jax.jit, static_argnames=())
def _run(features, labels, task_ids, perms):
    pool = pl.pallas_call(
        _pool_body,
        grid=(_B,),
        in_specs=[
            pl.BlockSpec((1, _D, 128, 128), lambda i: (i, 0, 0, 0)),
            pl.BlockSpec((1, 2, 128, 128), lambda i: (i, 0, 0, 0)),
        ],
        out_specs=[
            pl.BlockSpec((1, 2, _D), lambda i: (i, 0, 0)),
            pl.BlockSpec((1, 2, _D), lambda i: (i, 0, 0)),
            pl.BlockSpec((1, 2, _D), lambda i: (i, 0, 0)),
        ],
        out_shape=[
            jax.ShapeDtypeStruct((_B, 2, _D), jnp.float32),
            jax.ShapeDtypeStruct((_B, 2, _D), jnp.float32),
            jax.ShapeDtypeStruct((_B, 2, _D), jnp.float32),
        ],
    )
    t_sums, b_sums, cnts = pool(features, labels)

    t2 = t_sums.reshape(_N2, _D)
    b2 = b_sums.reshape(_N2, _D)
    cnt = cnts[:, :, 0].reshape(_N2)

    finale = pl.pallas_call(
        _finale_body,
        out_shape=jax.ShapeDtypeStruct((1, 1), jnp.float32),
    )
    loss = finale(
        t2, b2,
        cnt.reshape(_N2, 1), cnt.reshape(1, _N2),
        task_ids.reshape(_N2, 1), task_ids.reshape(1, _N2),
        perms,
    )
    return loss[0, 0]


_rng = np.random.default_rng(0)
_PERMS = np.stack(
    [_rng.permutation(_D)[:N_NEGATIVES] for _ in range(_N2)]).astype(np.int32)


def kernel(features, labels, tasks):
    task_ids = jnp.stack([2 * tasks, 2 * tasks + 1], axis=1).reshape(-1)
    return _run(features, labels, task_ids.astype(jnp.int32), _PERMS)
